# Initial kernel scaffold; baseline (speedup 1.0000x reference)
import jax
import jax.numpy as jnp
from jax.experimental import pallas as pl


def kernel(input_ids, values, word_emb, cat_fc_w, cat_fc_b, cat_ln_g, cat_ln_b, cat_proj_w, cat_proj_b, pos_emb, tok_emb, ln_g, ln_b):
    raise NotImplementedError("write your pallas kernel here")


# speedup vs baseline: 12.8118x; 12.8118x over previous
import functools

import jax
import jax.numpy as jnp
from jax import lax
from jax.experimental import pallas as pl
from jax.experimental.pallas import tpu as pltpu
from jax.experimental.pallas import tpu_sc as plsc

B, S, V, H = 1024, 200, 1000000, 128

NC, NS = 2, 16
NW = NC * NS
NUM_IDS = B * S
IDS_PER_W = NUM_IDS // NW
CHUNK = 128
NCHUNK = IDS_PER_W // CHUNK


def _gather_body(ids_hbm, table_hbm, out_hbm, ids_v, rows0, rows1, sem0, sem1):
    wid = lax.axis_index("s") * NC + lax.axis_index("c")
    out_base = wid * IDS_PER_W
    pltpu.sync_copy(ids_hbm.at[wid], ids_v)
    bufs = (rows0, rows1)
    sems = (sem0, sem1)
    pltpu.async_copy(table_hbm.at[ids_v.at[0]], rows0, sem0)
    pltpu.async_copy(table_hbm.at[ids_v.at[1]], rows1, sem1)

    @pl.loop(0, NCHUNK, step=2)
    def _(j0):
        for b in range(2):
            j = j0 + b
            buf, sem = bufs[b], sems[b]
            pltpu.make_async_copy(table_hbm.at[ids_v.at[j]], buf, sem).wait()
            pltpu.sync_copy(buf, out_hbm.at[pl.ds(out_base + j * CHUNK, CHUNK)])

            @pl.when(j + 2 < NCHUNK)
            def _():
                pltpu.async_copy(table_hbm.at[ids_v.at[j + 2]], buf, sem)


@functools.cache
def _sc_gather():
    return pl.kernel(
        _gather_body,
        out_type=jax.ShapeDtypeStruct((NUM_IDS, H), jnp.float32),
        mesh=plsc.VectorSubcoreMesh(core_axis_name="c", subcore_axis_name="s",
                                    num_cores=NC, num_subcores=NS),
        scratch_types=[
            pltpu.VMEM((NCHUNK, CHUNK), jnp.int32),
            pltpu.VMEM((CHUNK, H), jnp.float32),
            pltpu.VMEM((CHUNK, H), jnp.float32),
            pltpu.SemaphoreType.DMA,
            pltpu.SemaphoreType.DMA,
        ],
    )


RB = 16
NBLK = B // RB


def _mlp_body(x_ref, v_ref, pe_ref, tok_ref, w1a_ref, w1b_ref, b1_ref,
              g1_ref, bb1_ref, w2_ref, b2_ref, g2_ref, bb2_ref, o_ref):
    x = x_ref[...].reshape(RB * S, H)
    v = v_ref[...].reshape(RB * S, 1)
    h = jnp.dot(x, w1a_ref[...], preferred_element_type=jnp.float32)
    h = h + v * w1b_ref[...].reshape(1, H) + b1_ref[...].reshape(1, H)
    m = h.mean(-1, keepdims=True)
    var = ((h - m) ** 2).mean(-1, keepdims=True)
    h = (h - m) * lax.rsqrt(var + 1e-5)
    h = h * g1_ref[...].reshape(1, H) + bb1_ref[...].reshape(1, H)
    h = h * jax.nn.sigmoid(1.702 * h)
    h = jnp.dot(h, w2_ref[...], preferred_element_type=jnp.float32)
    h = h + b2_ref[...].reshape(1, H)
    emb = h.reshape(RB, S, H) + pe_ref[...][None] + tok_ref[...][0][None, None]
    m2 = emb.mean(-1, keepdims=True)
    var2 = ((emb - m2) ** 2).mean(-1, keepdims=True)
    o_ref[...] = (emb - m2) * lax.rsqrt(var2 + 1e-12) \
        * g2_ref[...].reshape(1, 1, H) + bb2_ref[...].reshape(1, 1, H)


def _const_spec(shape):
    return pl.BlockSpec(shape, lambda i: tuple(0 for _ in shape))


_tc_mlp = pl.pallas_call(
    _mlp_body,
    grid=(NBLK,),
    in_specs=[
        pl.BlockSpec((RB, S, H), lambda i: (i, 0, 0)),
        pl.BlockSpec((RB, S, 1), lambda i: (i, 0, 0)),
        _const_spec((S, H)),
        _const_spec((2, H)),
        _const_spec((H, H)),
        _const_spec((H,)),
        _const_spec((H,)),
        _const_spec((H,)),
        _const_spec((H,)),
        _const_spec((H, H)),
        _const_spec((H,)),
        _const_spec((H,)),
        _const_spec((H,)),
    ],
    out_specs=pl.BlockSpec((RB, S, H), lambda i: (i, 0, 0)),
    out_shape=jax.ShapeDtypeStruct((B, S, H), jnp.float32),
    compiler_params=pltpu.CompilerParams(
        dimension_semantics=("arbitrary",),
    ),
)


def kernel(input_ids, values, word_emb, cat_fc_w, cat_fc_b, cat_ln_g, cat_ln_b,
           cat_proj_w, cat_proj_b, pos_emb, tok_emb, ln_g, ln_b):
    ids = input_ids.astype(jnp.int32).reshape(NW, NCHUNK, CHUNK)
    gathered = _sc_gather()(ids, word_emb)
    out = _tc_mlp(
        gathered.reshape(B, S, H),
        values.astype(jnp.float32).reshape(B, S, 1),
        pos_emb[:S],
        tok_emb,
        cat_fc_w[:H],
        cat_fc_w[H],
        cat_fc_b,
        cat_ln_g,
        cat_ln_b,
        cat_proj_w,
        cat_proj_b,
        ln_g,
        ln_b,
    )
    return out
